# baseline (speedup 1.0000x reference)
import functools
import math

import jax
import jax.numpy as jnp
from jax import lax
from jax.experimental import pallas as pl
from jax.experimental.pallas import tpu as pltpu
from jax.experimental.pallas import tpu_sc as plsc

N_NODES = 10000
N_EDGES = 160000
R_CUT = 5.0
N_RBF = 20

NC = 2
NS = 16
BATCH = 128
NB = 80
EPT = NB * BATCH
E_PAD = NS * EPT
NODES_PAD = 10112
ZROWS = NODES_PAD // NS

B_BLK = 1280
N_REAL_BLK = N_EDGES // B_BLK


def _norm_body(r_ref, o_ref):
    x = r_ref[...]
    o_ref[0, 0] = 1.0 / jnp.sqrt(jnp.sum(x * x))


def _dense_body(inv_ref, s_ref, r_ref, v_ref, w1_ref, b1_ref, w2_ref,
                b2_ref, wr_ref, br_ref, o_ref):
    s = s_ref[...]
    h = jnp.dot(s, w1_ref[...], preferred_element_type=jnp.float32)
    h = h + b1_ref[...]
    h = h * jax.nn.sigmoid(h)
    phi = jnp.dot(h, w2_ref[...], preferred_element_type=jnp.float32)
    phi = phi + b2_ref[...]

    r = r_ref[...]
    rn = jnp.sqrt(jnp.sum(r * r, axis=1, keepdims=True))
    n_vals = (lax.broadcasted_iota(jnp.int32, (1, N_RBF), 1) + 1).astype(
        jnp.float32) * (math.pi / R_CUT)
    rb = jnp.sin(rn * n_vals) / rn
    rb = jnp.where(rb <= R_CUT, 0.5 * (jnp.cos(rb * (math.pi / R_CUT)) + 1.0),
                   0.0)
    w = jnp.dot(rb, wr_ref[...], preferred_element_type=jnp.float32)
    w = w + br_ref[...]

    split = w * phi
    s0 = split[:, 0:128]
    s1 = split[:, 128:256]
    s2 = split[:, 256:384]
    inv = inv_ref[0, 0]
    for d in range(3):
        o_ref[d] = s0 * v_ref[:, d * 128:(d + 1) * 128] + s2 * (
            r[:, d:d + 1] * inv)
    o_ref[3] = s1


def _scatter_body(msg_hbm, idx_hbm, zeros_hbm, out_hbm,
                  idx_v, buf0, buf1, acc, sem0, sem1):
    c = lax.axis_index("c")
    t = lax.axis_index("s")
    base = t * EPT

    pltpu.sync_copy(idx_hbm.at[t], idx_v)

    bufs = (buf0, buf1)
    sems = (sem0, sem1)

    for ci in range(2):
        chunk = c * 2 + ci
        pltpu.sync_copy(zeros_hbm, acc.at[pl.ds(t * ZROWS, ZROWS)])
        plsc.subcore_barrier()

        def body(j, _):
            pltpu.sync_copy(
                msg_hbm.at[chunk].at[pl.ds(base + j * BATCH, BATCH)], buf0)
            pltpu.sync_copy(buf0, acc.at[idx_v.at[j]], add=True)
            return 0

        lax.fori_loop(0, NB, body, 0)
        plsc.subcore_barrier()

        pltpu.sync_copy(acc.at[pl.ds(t * ZROWS, ZROWS)],
                        out_hbm.at[chunk].at[pl.ds(t * ZROWS, ZROWS)])
        plsc.subcore_barrier()


def _finish_body(a3_ref, a1_ref, ov_ref, os_ref):
    for d in range(3):
        ov_ref[:, d, :] = a3_ref[d]
    os_ref[:, 0, :] = a1_ref[0]


@functools.lru_cache(maxsize=1)
def _get_scatter_call():
  return functools.partial(
    pl.kernel,
    out_type=jax.ShapeDtypeStruct((4, NODES_PAD, 128), jnp.float32),
    mesh=plsc.VectorSubcoreMesh(
        core_axis_name="c", subcore_axis_name="s", num_cores=NC,
        num_subcores=NS),
    scratch_types=[
        pltpu.VMEM((NB, BATCH), jnp.int32),
        pltpu.VMEM((BATCH, 128), jnp.float32),
        pltpu.VMEM((BATCH, 128), jnp.float32),
        pltpu.VMEM_SHARED((NODES_PAD, 128), jnp.float32),
        pltpu.SemaphoreType.DMA,
        pltpu.SemaphoreType.DMA,
    ],
    compiler_params=pltpu.CompilerParams(use_tc_tiling_on_sc=True),
  )(_scatter_body)


@jax.jit
def kernel(s, r, v, idx_i, W1, b1, W2, b2, Wr, br):
    E = N_EDGES
    s2d = s.reshape(E, 128)
    v2d = v.reshape(E, 384)
    r2d = r.reshape(-1, 128)

    inv = pl.pallas_call(
        _norm_body,
        out_shape=jax.ShapeDtypeStruct((1, 1), jnp.float32),
        in_specs=[pl.BlockSpec((r2d.shape[0], 128), lambda: (0, 0))],
        out_specs=pl.BlockSpec(memory_space=pltpu.SMEM),
    )(r2d)

    grid = (E_PAD // B_BLK,)
    msg = pl.pallas_call(
        _dense_body,
        grid=grid,
        in_specs=[
            pl.BlockSpec(memory_space=pltpu.SMEM),
            pl.BlockSpec((B_BLK, 128),
                         lambda i: (jnp.minimum(i, N_REAL_BLK - 1), 0)),
            pl.BlockSpec((B_BLK, 3),
                         lambda i: (jnp.minimum(i, N_REAL_BLK - 1), 0)),
            pl.BlockSpec((B_BLK, 384),
                         lambda i: (jnp.minimum(i, N_REAL_BLK - 1), 0)),
            pl.BlockSpec((128, 128), lambda i: (0, 0)),
            pl.BlockSpec((1, 128), lambda i: (0, 0)),
            pl.BlockSpec((128, 384), lambda i: (0, 0)),
            pl.BlockSpec((1, 384), lambda i: (0, 0)),
            pl.BlockSpec((N_RBF, 384), lambda i: (0, 0)),
            pl.BlockSpec((1, 384), lambda i: (0, 0)),
        ],
        out_specs=pl.BlockSpec((4, B_BLK, 128), lambda i: (0, i, 0)),
        out_shape=jax.ShapeDtypeStruct((4, E_PAD, 128), jnp.float32),
        compiler_params=pltpu.CompilerParams(
            dimension_semantics=("arbitrary",)),
    )(inv, s2d, r, v2d, W1, b1.reshape(1, 128), W2, b2.reshape(1, 384),
      Wr, br.reshape(1, 384))

    idx_pad = jnp.concatenate(
        [idx_i, jnp.full((E_PAD - E,), N_NODES, dtype=jnp.int32)]
    ).reshape(NS, NB, BATCH)
    zeros = jnp.zeros((ZROWS, 128), jnp.float32)

    out4 = _get_scatter_call()(msg, idx_pad, zeros)

    FN = 400
    out_v, out_s = pl.pallas_call(
        _finish_body,
        grid=(N_NODES // FN,),
        in_specs=[
            pl.BlockSpec((3, FN, 128), lambda i: (0, i, 0)),
            pl.BlockSpec((1, FN, 128), lambda i: (3, i, 0)),
        ],
        out_specs=[
            pl.BlockSpec((FN, 3, 128), lambda i: (i, 0, 0)),
            pl.BlockSpec((FN, 1, 128), lambda i: (i, 0, 0)),
        ],
        out_shape=[
            jax.ShapeDtypeStruct((N_NODES, 3, 128), jnp.float32),
            jax.ShapeDtypeStruct((N_NODES, 1, 128), jnp.float32),
        ],
        compiler_params=pltpu.CompilerParams(
            dimension_semantics=("arbitrary",)),
    )(out4, out4)
    return (out_v, out_s)


# speedup vs baseline: 1.3305x; 1.3305x over previous
import functools
import math

import jax
import jax.numpy as jnp
from jax import lax
from jax.experimental import pallas as pl
from jax.experimental.pallas import tpu as pltpu
from jax.experimental.pallas import tpu_sc as plsc

N_NODES = 10000
N_EDGES = 160000
R_CUT = 5.0
N_RBF = 20

NC = 2
NS = 16
BATCH = 128
NB = 80
EPT = NB * BATCH
E_PAD = NS * EPT
NODES_PAD = 10112
ZROWS = NODES_PAD // NS

B_BLK = 1280
N_REAL_BLK = N_EDGES // B_BLK


def _norm_body(r_ref, o_ref):
    x = r_ref[...]
    o_ref[0, 0] = 1.0 / jnp.sqrt(jnp.sum(x * x))


def _dense_body(inv_ref, s_ref, r_ref, v_ref, w1_ref, b1_ref, w2_ref,
                b2_ref, wr_ref, br_ref, o_ref):
    s = s_ref[...]
    h = jnp.dot(s, w1_ref[...], preferred_element_type=jnp.float32)
    h = h + b1_ref[...]
    h = h * jax.nn.sigmoid(h)
    phi = jnp.dot(h, w2_ref[...], preferred_element_type=jnp.float32)
    phi = phi + b2_ref[...]

    r = r_ref[...]
    rn = jnp.sqrt(jnp.sum(r * r, axis=1, keepdims=True))
    n_vals = (lax.broadcasted_iota(jnp.int32, (1, N_RBF), 1) + 1).astype(
        jnp.float32) * (math.pi / R_CUT)
    rb = jnp.sin(rn * n_vals) / rn
    rb = jnp.where(rb <= R_CUT, 0.5 * (jnp.cos(rb * (math.pi / R_CUT)) + 1.0),
                   0.0)
    w = jnp.dot(rb, wr_ref[...], preferred_element_type=jnp.float32)
    w = w + br_ref[...]

    split = w * phi
    s0 = split[:, 0:128]
    s1 = split[:, 128:256]
    s2 = split[:, 256:384]
    inv = inv_ref[0, 0]
    for d in range(3):
        o_ref[d] = s0 * v_ref[d] + s2 * (r[:, d:d + 1] * inv)
    o_ref[3] = s1


def _scatter_body(msg_hbm, idx_hbm, zeros_hbm, outv_hbm, outs_hbm,
                  idx_v, buf0, buf1, acc, sem0, sem1):
    c = lax.axis_index("c")
    t = lax.axis_index("s")
    base = t * EPT

    pltpu.sync_copy(idx_hbm.at[t], idx_v)

    bufs = (buf0, buf1)
    sems = (sem0, sem1)

    for ci in range(2):
        chunk = c * 2 + ci
        pltpu.sync_copy(zeros_hbm, acc.at[pl.ds(t * ZROWS, ZROWS)])
        plsc.subcore_barrier()

        def body(j, _):
            pltpu.sync_copy(
                msg_hbm.at[chunk].at[pl.ds(base + j * BATCH, BATCH)], buf0)
            pltpu.sync_copy(buf0, acc.at[idx_v.at[j]], add=True)
            return 0

        lax.fori_loop(0, NB, body, 0)
        plsc.subcore_barrier()

        start = lax.min(t * ZROWS, N_NODES - ZROWS)

        @pl.when(chunk < 3)
        def _():
            pltpu.sync_copy(acc.at[pl.ds(start, ZROWS)],
                            outv_hbm.at[chunk].at[pl.ds(start, ZROWS)])

        @pl.when(chunk == 3)
        def _():
            pltpu.sync_copy(acc.at[pl.ds(start, ZROWS)],
                            outs_hbm.at[pl.ds(start, ZROWS)])

        plsc.subcore_barrier()


@functools.lru_cache(maxsize=1)
def _get_scatter_call():
  return functools.partial(
    pl.kernel,
    out_type=[
        jax.ShapeDtypeStruct((3, N_NODES, 128), jnp.float32),
        jax.ShapeDtypeStruct((N_NODES, 128), jnp.float32),
    ],
    mesh=plsc.VectorSubcoreMesh(
        core_axis_name="c", subcore_axis_name="s", num_cores=NC,
        num_subcores=NS),
    scratch_types=[
        pltpu.VMEM((NB, BATCH), jnp.int32),
        pltpu.VMEM((BATCH, 128), jnp.float32),
        pltpu.VMEM((BATCH, 128), jnp.float32),
        pltpu.VMEM_SHARED((NODES_PAD, 128), jnp.float32),
        pltpu.SemaphoreType.DMA,
        pltpu.SemaphoreType.DMA,
    ],
    compiler_params=pltpu.CompilerParams(use_tc_tiling_on_sc=True),
  )(_scatter_body)


@jax.jit
def kernel(s, r, v, idx_i, W1, b1, W2, b2, Wr, br):
    E = N_EDGES
    s2d = s.reshape(E, 128)
    vT = v.transpose(1, 0, 2)
    r2d = r.reshape(-1, 128)

    inv = pl.pallas_call(
        _norm_body,
        out_shape=jax.ShapeDtypeStruct((1, 1), jnp.float32),
        in_specs=[pl.BlockSpec((r2d.shape[0], 128), lambda: (0, 0))],
        out_specs=pl.BlockSpec(memory_space=pltpu.SMEM),
    )(r2d)

    grid = (E_PAD // B_BLK,)
    msg = pl.pallas_call(
        _dense_body,
        grid=grid,
        in_specs=[
            pl.BlockSpec(memory_space=pltpu.SMEM),
            pl.BlockSpec((B_BLK, 128),
                         lambda i: (jnp.minimum(i, N_REAL_BLK - 1), 0)),
            pl.BlockSpec((B_BLK, 3),
                         lambda i: (jnp.minimum(i, N_REAL_BLK - 1), 0)),
            pl.BlockSpec((3, B_BLK, 128),
                         lambda i: (0, jnp.minimum(i, N_REAL_BLK - 1), 0)),
            pl.BlockSpec((128, 128), lambda i: (0, 0)),
            pl.BlockSpec((1, 128), lambda i: (0, 0)),
            pl.BlockSpec((128, 384), lambda i: (0, 0)),
            pl.BlockSpec((1, 384), lambda i: (0, 0)),
            pl.BlockSpec((N_RBF, 384), lambda i: (0, 0)),
            pl.BlockSpec((1, 384), lambda i: (0, 0)),
        ],
        out_specs=pl.BlockSpec((4, B_BLK, 128), lambda i: (0, i, 0)),
        out_shape=jax.ShapeDtypeStruct((4, E_PAD, 128), jnp.float32),
        compiler_params=pltpu.CompilerParams(
            dimension_semantics=("arbitrary",)),
    )(inv, s2d, r, vT, W1, b1.reshape(1, 128), W2, b2.reshape(1, 384),
      Wr, br.reshape(1, 384))

    idx_pad = jnp.concatenate(
        [idx_i, jnp.full((E_PAD - E,), N_NODES, dtype=jnp.int32)]
    ).reshape(NS, NB, BATCH)
    zeros = jnp.zeros((ZROWS, 128), jnp.float32)

    out_v3, out_s2 = _get_scatter_call()(msg, idx_pad, zeros)
    return (out_v3.transpose(1, 0, 2), out_s2.reshape(N_NODES, 1, 128))


# speedup vs baseline: 1.4070x; 1.0575x over previous
import functools
import math

import jax
import jax.numpy as jnp
from jax import lax
from jax.experimental import pallas as pl
from jax.experimental.pallas import tpu as pltpu
from jax.experimental.pallas import tpu_sc as plsc

N_NODES = 10000
N_EDGES = 160000
R_CUT = 5.0
N_RBF = 20

NC = 2
NS = 16
BATCH = 128
NB = 80
EPT = NB * BATCH
E_PAD = NS * EPT
NODES_PAD = 10112
ZROWS = NODES_PAD // NS

B_BLK = 1280
N_REAL_BLK = N_EDGES // B_BLK


def _norm_body(r_ref, o_ref):
    x = r_ref[...]
    o_ref[0, 0] = 1.0 / jnp.sqrt(jnp.sum(x * x))


def _dense_body(inv_ref, s_ref, r_ref, v_ref, w1_ref, b1_ref, w2_ref,
                b2_ref, wr_ref, br_ref, o_ref):
    s = s_ref[...]
    h = jnp.dot(s, w1_ref[...], preferred_element_type=jnp.float32)
    h = h + b1_ref[...]
    h = h * jax.nn.sigmoid(h)
    phi = jnp.dot(h, w2_ref[...], preferred_element_type=jnp.float32)
    phi = phi + b2_ref[...]

    r = jnp.transpose(r_ref[...], (1, 0))
    rn = jnp.sqrt(jnp.sum(r * r, axis=1, keepdims=True))
    n_vals = (lax.broadcasted_iota(jnp.int32, (1, N_RBF), 1) + 1).astype(
        jnp.float32) * (math.pi / R_CUT)
    rb = jnp.sin(rn * n_vals) / rn
    rb = jnp.where(rb <= R_CUT, 0.5 * (jnp.cos(rb * (math.pi / R_CUT)) + 1.0),
                   0.0)
    w = jnp.dot(rb, wr_ref[...], preferred_element_type=jnp.float32)
    w = w + br_ref[...]

    split = w * phi
    s0 = split[:, 0:128]
    s1 = split[:, 128:256]
    s2 = split[:, 256:384]
    inv = inv_ref[0, 0]
    for d in range(3):
        o_ref[d] = s0 * v_ref[d] + s2 * (r[:, d:d + 1] * inv)
    o_ref[3] = s1


def _scatter_body(msg_hbm, idx_hbm, zeros_hbm, outv_hbm, outs_hbm,
                  idx_v, buf0, buf1, acc, sem0, sem1):
    c = lax.axis_index("c")
    t = lax.axis_index("s")
    base = t * EPT

    pltpu.sync_copy(idx_hbm.at[t], idx_v)

    bufs = (buf0, buf1)
    sems = (sem0, sem1)

    for ci in range(2):
        chunk = c * 2 + ci
        pltpu.sync_copy(zeros_hbm, acc.at[pl.ds(t * ZROWS, ZROWS)])
        plsc.subcore_barrier()

        def body(j, _):
            pltpu.sync_copy(
                msg_hbm.at[chunk].at[pl.ds(base + j * BATCH, BATCH)], buf0)
            pltpu.sync_copy(buf0, acc.at[idx_v.at[j]], add=True)
            return 0

        lax.fori_loop(0, NB, body, 0)
        plsc.subcore_barrier()

        start = lax.min(t * ZROWS, N_NODES - ZROWS)

        @pl.when(chunk < 3)
        def _():
            pltpu.sync_copy(acc.at[pl.ds(start, ZROWS)],
                            outv_hbm.at[chunk].at[pl.ds(start, ZROWS)])

        @pl.when(chunk == 3)
        def _():
            pltpu.sync_copy(acc.at[pl.ds(start, ZROWS)],
                            outs_hbm.at[pl.ds(start, ZROWS)])

        plsc.subcore_barrier()


@functools.lru_cache(maxsize=1)
def _get_scatter_call():
  return functools.partial(
    pl.kernel,
    out_type=[
        jax.ShapeDtypeStruct((3, N_NODES, 128), jnp.float32),
        jax.ShapeDtypeStruct((N_NODES, 128), jnp.float32),
    ],
    mesh=plsc.VectorSubcoreMesh(
        core_axis_name="c", subcore_axis_name="s", num_cores=NC,
        num_subcores=NS),
    scratch_types=[
        pltpu.VMEM((NB, BATCH), jnp.int32),
        pltpu.VMEM((BATCH, 128), jnp.float32),
        pltpu.VMEM((BATCH, 128), jnp.float32),
        pltpu.VMEM_SHARED((NODES_PAD, 128), jnp.float32),
        pltpu.SemaphoreType.DMA,
        pltpu.SemaphoreType.DMA,
    ],
    compiler_params=pltpu.CompilerParams(use_tc_tiling_on_sc=True),
  )(_scatter_body)


@jax.jit
def kernel(s, r, v, idx_i, W1, b1, W2, b2, Wr, br):
    E = N_EDGES
    s2d = s.reshape(E, 128)
    vT = v.transpose(1, 0, 2)
    rT = r.transpose(1, 0)
    r2d = rT.reshape(-1, 128)

    inv = pl.pallas_call(
        _norm_body,
        out_shape=jax.ShapeDtypeStruct((1, 1), jnp.float32),
        in_specs=[pl.BlockSpec((r2d.shape[0], 128), lambda: (0, 0))],
        out_specs=pl.BlockSpec(memory_space=pltpu.SMEM),
    )(r2d)

    grid = (E_PAD // B_BLK,)
    msg = pl.pallas_call(
        _dense_body,
        grid=grid,
        in_specs=[
            pl.BlockSpec(memory_space=pltpu.SMEM),
            pl.BlockSpec((B_BLK, 128),
                         lambda i: (jnp.minimum(i, N_REAL_BLK - 1), 0)),
            pl.BlockSpec((3, B_BLK),
                         lambda i: (0, jnp.minimum(i, N_REAL_BLK - 1))),
            pl.BlockSpec((3, B_BLK, 128),
                         lambda i: (0, jnp.minimum(i, N_REAL_BLK - 1), 0)),
            pl.BlockSpec((128, 128), lambda i: (0, 0)),
            pl.BlockSpec((1, 128), lambda i: (0, 0)),
            pl.BlockSpec((128, 384), lambda i: (0, 0)),
            pl.BlockSpec((1, 384), lambda i: (0, 0)),
            pl.BlockSpec((N_RBF, 384), lambda i: (0, 0)),
            pl.BlockSpec((1, 384), lambda i: (0, 0)),
        ],
        out_specs=pl.BlockSpec((4, B_BLK, 128), lambda i: (0, i, 0)),
        out_shape=jax.ShapeDtypeStruct((4, E_PAD, 128), jnp.float32),
        compiler_params=pltpu.CompilerParams(
            dimension_semantics=("arbitrary",)),
    )(inv, s2d, rT, vT, W1, b1.reshape(1, 128), W2,
      b2.reshape(1, 384), Wr, br.reshape(1, 384))

    idx_pad = jnp.concatenate(
        [idx_i, jnp.full((E_PAD - E,), N_NODES, dtype=jnp.int32)]
    ).reshape(NS, NB, BATCH)
    zeros = jnp.zeros((ZROWS, 128), jnp.float32)

    out_v3, out_s2 = _get_scatter_call()(msg, idx_pad, zeros)
    return (out_v3.transpose(1, 0, 2), out_s2.reshape(N_NODES, 1, 128))


# speedup vs baseline: 1.5994x; 1.1367x over previous
import functools
import math

import jax
import jax.numpy as jnp
from jax import lax
from jax.experimental import pallas as pl
from jax.experimental.pallas import tpu as pltpu
from jax.experimental.pallas import tpu_sc as plsc

N_NODES = 10000
N_EDGES = 160000
R_CUT = 5.0
N_RBF = 20

NC = 2
NS = 16
BATCH = 128
NB = 80
EPT = NB * BATCH
E_PAD = NS * EPT
NODES_PAD = 10112
ZROWS = NODES_PAD // NS

B_BLK = 1280
N_REAL_BLK = N_EDGES // B_BLK


def _norm_body(r_ref, o_ref):
    x = r_ref[...]
    o_ref[0, 0] = 1.0 / jnp.sqrt(jnp.sum(x * x))


def _dense_body(inv_ref, s_ref, r_ref, v_ref, w1_ref, b1_ref, w2_ref,
                b2_ref, wr_ref, br_ref, o_ref):
    s = s_ref[...]
    h = jnp.dot(s, w1_ref[...], preferred_element_type=jnp.float32)
    h = h + b1_ref[...]
    h = h * jax.nn.sigmoid(h)
    phi = jnp.dot(h, w2_ref[...], preferred_element_type=jnp.float32)
    phi = phi + b2_ref[...]

    r = jnp.transpose(r_ref[...], (1, 0))
    rn = jnp.sqrt(jnp.sum(r * r, axis=1, keepdims=True))
    n_vals = (lax.broadcasted_iota(jnp.int32, (1, N_RBF), 1) + 1).astype(
        jnp.float32) * (math.pi / R_CUT)
    rb = jnp.sin(rn * n_vals) / rn
    rb = jnp.where(rb <= R_CUT, 0.5 * (jnp.cos(rb * (math.pi / R_CUT)) + 1.0),
                   0.0)
    w = jnp.dot(rb, wr_ref[...], preferred_element_type=jnp.float32)
    w = w + br_ref[...]

    split = w * phi
    s0 = split[:, 0:128]
    s1 = split[:, 128:256]
    s2 = split[:, 256:384]
    inv = inv_ref[0, 0]
    for d in range(3):
        o_ref[d] = s0 * v_ref[d] + s2 * (r[:, d:d + 1] * inv)
    o_ref[3] = s1


def _scatter_body(msg_hbm, idx_hbm, zeros_hbm, outv_hbm, outs_hbm,
                  idx_v, buf0, buf1, acc, sem0, sem1):
    c = lax.axis_index("c")
    t = lax.axis_index("s")
    base = t * EPT

    pltpu.sync_copy(idx_hbm.at[t], idx_v)

    bufs = (buf0, buf1)
    sems = (sem0, sem1)

    for ci in range(2):
        chunk = c * 2 + ci
        pltpu.sync_copy(zeros_hbm, acc.at[pl.ds(t * ZROWS, ZROWS)])
        plsc.subcore_barrier()

        for h in range(2):
            pltpu.async_copy(
                msg_hbm.at[chunk].at[pl.ds(base + h * BATCH, BATCH)],
                bufs[h], sems[h])

        def body(j2, _):
            for h in range(2):
                j = j2 * 2 + h
                pltpu.make_async_copy(
                    msg_hbm.at[chunk].at[pl.ds(base, BATCH)],
                    bufs[h], sems[h]).wait()
                pltpu.sync_copy(bufs[h], acc.at[idx_v.at[j]], add=True)
                jn = lax.min(j + 2, NB - 1)
                pltpu.async_copy(
                    msg_hbm.at[chunk].at[pl.ds(base + jn * BATCH, BATCH)],
                    bufs[h], sems[h])
            return 0

        lax.fori_loop(0, NB // 2, body, 0)
        for h in range(2):
            pltpu.make_async_copy(
                msg_hbm.at[chunk].at[pl.ds(base, BATCH)],
                bufs[h], sems[h]).wait()
        plsc.subcore_barrier()

        start = lax.min(t * ZROWS, N_NODES - ZROWS)

        @pl.when(chunk < 3)
        def _():
            pltpu.sync_copy(acc.at[pl.ds(start, ZROWS)],
                            outv_hbm.at[chunk].at[pl.ds(start, ZROWS)])

        @pl.when(chunk == 3)
        def _():
            pltpu.sync_copy(acc.at[pl.ds(start, ZROWS)],
                            outs_hbm.at[pl.ds(start, ZROWS)])

        plsc.subcore_barrier()


@functools.lru_cache(maxsize=1)
def _get_scatter_call():
  return functools.partial(
    pl.kernel,
    out_type=[
        jax.ShapeDtypeStruct((3, N_NODES, 128), jnp.float32),
        jax.ShapeDtypeStruct((N_NODES, 128), jnp.float32),
    ],
    mesh=plsc.VectorSubcoreMesh(
        core_axis_name="c", subcore_axis_name="s", num_cores=NC,
        num_subcores=NS),
    scratch_types=[
        pltpu.VMEM((NB, BATCH), jnp.int32),
        pltpu.VMEM((BATCH, 128), jnp.float32),
        pltpu.VMEM((BATCH, 128), jnp.float32),
        pltpu.VMEM_SHARED((NODES_PAD, 128), jnp.float32),
        pltpu.SemaphoreType.DMA,
        pltpu.SemaphoreType.DMA,
    ],
    compiler_params=pltpu.CompilerParams(use_tc_tiling_on_sc=True),
  )(_scatter_body)


@jax.jit
def kernel(s, r, v, idx_i, W1, b1, W2, b2, Wr, br):
    E = N_EDGES
    s2d = s.reshape(E, 128)
    vT = v.transpose(1, 0, 2)
    rT = r.transpose(1, 0)
    r2d = rT.reshape(-1, 128)

    inv = pl.pallas_call(
        _norm_body,
        out_shape=jax.ShapeDtypeStruct((1, 1), jnp.float32),
        in_specs=[pl.BlockSpec((r2d.shape[0], 128), lambda: (0, 0))],
        out_specs=pl.BlockSpec(memory_space=pltpu.SMEM),
    )(r2d)

    grid = (E_PAD // B_BLK,)
    msg = pl.pallas_call(
        _dense_body,
        grid=grid,
        in_specs=[
            pl.BlockSpec(memory_space=pltpu.SMEM),
            pl.BlockSpec((B_BLK, 128),
                         lambda i: (jnp.minimum(i, N_REAL_BLK - 1), 0)),
            pl.BlockSpec((3, B_BLK),
                         lambda i: (0, jnp.minimum(i, N_REAL_BLK - 1))),
            pl.BlockSpec((3, B_BLK, 128),
                         lambda i: (0, jnp.minimum(i, N_REAL_BLK - 1), 0)),
            pl.BlockSpec((128, 128), lambda i: (0, 0)),
            pl.BlockSpec((1, 128), lambda i: (0, 0)),
            pl.BlockSpec((128, 384), lambda i: (0, 0)),
            pl.BlockSpec((1, 384), lambda i: (0, 0)),
            pl.BlockSpec((N_RBF, 384), lambda i: (0, 0)),
            pl.BlockSpec((1, 384), lambda i: (0, 0)),
        ],
        out_specs=pl.BlockSpec((4, B_BLK, 128), lambda i: (0, i, 0)),
        out_shape=jax.ShapeDtypeStruct((4, E_PAD, 128), jnp.float32),
        compiler_params=pltpu.CompilerParams(
            dimension_semantics=("arbitrary",)),
    )(inv, s2d, rT, vT, W1, b1.reshape(1, 128), W2,
      b2.reshape(1, 384), Wr, br.reshape(1, 384))

    idx_pad = jnp.concatenate(
        [idx_i, jnp.full((E_PAD - E,), N_NODES, dtype=jnp.int32)]
    ).reshape(NS, NB, BATCH)
    zeros = jnp.zeros((ZROWS, 128), jnp.float32)

    out_v3, out_s2 = _get_scatter_call()(msg, idx_pad, zeros)
    return (out_v3.transpose(1, 0, 2), out_s2.reshape(N_NODES, 1, 128))


# speedup vs baseline: 2.7101x; 1.6945x over previous
import functools
import math

import jax
import jax.numpy as jnp
from jax import lax
from jax.experimental import pallas as pl
from jax.experimental.pallas import tpu as pltpu
from jax.experimental.pallas import tpu_sc as plsc

N_NODES = 10000
N_EDGES = 160000
R_CUT = 5.0
N_RBF = 20

NC = 2
NS = 16
BATCH = 128
NB = 80
EPT = NB * BATCH
E_PAD = NS * EPT
NODES_PAD = 10112
ZROWS = NODES_PAD // NS

B_BLK = 1280
N_REAL_BLK = N_EDGES // B_BLK


def _norm_body(r_ref, o_ref):
    x = r_ref[...]
    o_ref[0, 0] = 1.0 / jnp.sqrt(jnp.sum(x * x))


def _dense_body(inv_ref, s_ref, r_ref, v_ref, w1_ref, b1_ref, w2_ref,
                b2_ref, wr_ref, br_ref, o_ref):
    s = s_ref[...]
    h = jnp.dot(s, w1_ref[...], preferred_element_type=jnp.float32)
    h = h + b1_ref[...]
    h = h * jax.nn.sigmoid(h)
    phi = jnp.dot(h, w2_ref[...], preferred_element_type=jnp.float32)
    phi = phi + b2_ref[...]

    rT = r_ref[...]
    rn_row = jnp.sqrt(jnp.sum(rT * rT, axis=0, keepdims=True))
    n_col = (lax.broadcasted_iota(jnp.int32, (N_RBF, 1), 0) + 1).astype(
        jnp.float32) * (math.pi / R_CUT)
    rbT = jnp.sin(n_col * rn_row) / rn_row
    rbT = jnp.where(rbT <= R_CUT,
                    0.5 * (jnp.cos(rbT * (math.pi / R_CUT)) + 1.0), 0.0)
    w = lax.dot_general(rbT, wr_ref[...], (((0,), (0,)), ((), ())),
                        preferred_element_type=jnp.float32)
    w = w + br_ref[...]

    r = jnp.transpose(rT, (1, 0))

    split = w * phi
    s0 = split[:, 0:128]
    s1 = split[:, 128:256]
    s2 = split[:, 256:384]
    inv = inv_ref[0, 0]
    for d in range(3):
        o_ref[d] = s0 * v_ref[d] + s2 * (r[:, d:d + 1] * inv)
    o_ref[3] = s1


def _scatter_body(msg_hbm, idx_hbm, zeros_hbm, outv_hbm, outs_hbm,
                  idx_v, buf0, buf1, acc, sem0, sem1):
    c = lax.axis_index("c")
    t = lax.axis_index("s")
    base = t * EPT

    pltpu.sync_copy(idx_hbm.at[t], idx_v)

    bufs = (buf0, buf1)
    sems = (sem0, sem1)

    for ci in range(2):
        chunk = c * 2 + ci
        pltpu.sync_copy(zeros_hbm, acc.at[pl.ds(t * ZROWS, ZROWS)])
        plsc.subcore_barrier()

        for h in range(2):
            pltpu.async_copy(
                msg_hbm.at[chunk].at[pl.ds(base + h * BATCH, BATCH)],
                bufs[h], sems[h])

        def body(j2, _):
            for h in range(2):
                j = j2 * 2 + h
                pltpu.make_async_copy(
                    msg_hbm.at[chunk].at[pl.ds(base, BATCH)],
                    bufs[h], sems[h]).wait()
                pltpu.sync_copy(bufs[h], acc.at[idx_v.at[j]], add=True)
                jn = lax.min(j + 2, NB - 1)
                pltpu.async_copy(
                    msg_hbm.at[chunk].at[pl.ds(base + jn * BATCH, BATCH)],
                    bufs[h], sems[h])
            return 0

        lax.fori_loop(0, NB // 2, body, 0)
        for h in range(2):
            pltpu.make_async_copy(
                msg_hbm.at[chunk].at[pl.ds(base, BATCH)],
                bufs[h], sems[h]).wait()
        plsc.subcore_barrier()

        start = lax.min(t * ZROWS, N_NODES - ZROWS)

        @pl.when(chunk < 3)
        def _():
            pltpu.sync_copy(acc.at[pl.ds(start, ZROWS)],
                            outv_hbm.at[chunk].at[pl.ds(start, ZROWS)])

        @pl.when(chunk == 3)
        def _():
            pltpu.sync_copy(acc.at[pl.ds(start, ZROWS)],
                            outs_hbm.at[pl.ds(start, ZROWS)])

        plsc.subcore_barrier()


@functools.lru_cache(maxsize=1)
def _get_scatter_call():
  return functools.partial(
    pl.kernel,
    out_type=[
        jax.ShapeDtypeStruct((3, N_NODES, 128), jnp.float32),
        jax.ShapeDtypeStruct((N_NODES, 128), jnp.float32),
    ],
    mesh=plsc.VectorSubcoreMesh(
        core_axis_name="c", subcore_axis_name="s", num_cores=NC,
        num_subcores=NS),
    scratch_types=[
        pltpu.VMEM((NB, BATCH), jnp.int32),
        pltpu.VMEM((BATCH, 128), jnp.float32),
        pltpu.VMEM((BATCH, 128), jnp.float32),
        pltpu.VMEM_SHARED((NODES_PAD, 128), jnp.float32),
        pltpu.SemaphoreType.DMA,
        pltpu.SemaphoreType.DMA,
    ],
    compiler_params=pltpu.CompilerParams(use_tc_tiling_on_sc=True),
  )(_scatter_body)


@jax.jit
def kernel(s, r, v, idx_i, W1, b1, W2, b2, Wr, br):
    E = N_EDGES
    s2d = s.reshape(E, 128)
    vT = v.transpose(1, 0, 2)
    rT = r.transpose(1, 0)
    r2d = rT.reshape(-1, 128)

    inv = pl.pallas_call(
        _norm_body,
        out_shape=jax.ShapeDtypeStruct((1, 1), jnp.float32),
        in_specs=[pl.BlockSpec((r2d.shape[0], 128), lambda: (0, 0))],
        out_specs=pl.BlockSpec(memory_space=pltpu.SMEM),
    )(r2d)

    grid = (E_PAD // B_BLK,)
    msg = pl.pallas_call(
        _dense_body,
        grid=grid,
        in_specs=[
            pl.BlockSpec(memory_space=pltpu.SMEM),
            pl.BlockSpec((B_BLK, 128),
                         lambda i: (jnp.minimum(i, N_REAL_BLK - 1), 0)),
            pl.BlockSpec((3, B_BLK),
                         lambda i: (0, jnp.minimum(i, N_REAL_BLK - 1))),
            pl.BlockSpec((3, B_BLK, 128),
                         lambda i: (0, jnp.minimum(i, N_REAL_BLK - 1), 0)),
            pl.BlockSpec((128, 128), lambda i: (0, 0)),
            pl.BlockSpec((1, 128), lambda i: (0, 0)),
            pl.BlockSpec((128, 384), lambda i: (0, 0)),
            pl.BlockSpec((1, 384), lambda i: (0, 0)),
            pl.BlockSpec((N_RBF, 384), lambda i: (0, 0)),
            pl.BlockSpec((1, 384), lambda i: (0, 0)),
        ],
        out_specs=pl.BlockSpec((4, B_BLK, 128), lambda i: (0, i, 0)),
        out_shape=jax.ShapeDtypeStruct((4, E_PAD, 128), jnp.float32),
        compiler_params=pltpu.CompilerParams(
            dimension_semantics=("arbitrary",)),
    )(inv, s2d, rT, vT, W1, b1.reshape(1, 128), W2,
      b2.reshape(1, 384), Wr, br.reshape(1, 384))

    idx_pad = jnp.concatenate(
        [idx_i, jnp.full((E_PAD - E,), N_NODES, dtype=jnp.int32)]
    ).reshape(NS, NB, BATCH)
    zeros = jnp.zeros((ZROWS, 128), jnp.float32)

    out_v3, out_s2 = _get_scatter_call()(msg, idx_pad, zeros)
    return (out_v3.transpose(1, 0, 2), out_s2.reshape(N_NODES, 1, 128))
